# baseline (speedup 1.0000x reference)
import jax
import jax.numpy as jnp
from jax.experimental import pallas as pl
from jax.experimental.pallas import tpu as pltpu

_NCHUNKS = 8


def _dma_body(x_hbm, o_hbm, *sems):
    rows = x_hbm.shape[0]
    chunk = rows // _NCHUNKS
    copies = [
        pltpu.make_async_copy(
            x_hbm.at[pl.ds(i * chunk, chunk), :],
            o_hbm.at[pl.ds(i * chunk, chunk), :],
            sems[i],
        )
        for i in range(_NCHUNKS)
    ]
    for c in copies:
        c.start()
    for c in copies:
        c.wait()


def kernel(x):
    return pl.pallas_call(
        _dma_body,
        in_specs=[pl.BlockSpec(memory_space=pl.ANY)],
        out_specs=pl.BlockSpec(memory_space=pl.ANY),
        out_shape=jax.ShapeDtypeStruct(x.shape, x.dtype),
        scratch_shapes=[pltpu.SemaphoreType.DMA] * _NCHUNKS,
    )(x)


# speedup vs baseline: 43.3013x; 43.3013x over previous
import jax
import jax.numpy as jnp
from jax.experimental import pallas as pl


def _copy_body(x_ref, o_ref):
    o_ref[...] = x_ref[...]


def kernel(x):
    rows, cols = x.shape
    block_rows = 256
    grid = (rows // block_rows,)
    return pl.pallas_call(
        _copy_body,
        grid=grid,
        in_specs=[pl.BlockSpec((block_rows, cols), lambda i: (i, 0))],
        out_specs=pl.BlockSpec((block_rows, cols), lambda i: (i, 0)),
        out_shape=jax.ShapeDtypeStruct((rows, cols), x.dtype),
    )(x)


# speedup vs baseline: 49.0134x; 1.1319x over previous
import jax
import jax.numpy as jnp
from jax.experimental import pallas as pl


def _copy_body(x_ref, o_ref):
    o_ref[...] = x_ref[...]


def kernel(x):
    rows, cols = x.shape
    block_rows = 1024
    grid = (rows // block_rows,)
    return pl.pallas_call(
        _copy_body,
        grid=grid,
        in_specs=[pl.BlockSpec((block_rows, cols), lambda i: (i, 0))],
        out_specs=pl.BlockSpec((block_rows, cols), lambda i: (i, 0)),
        out_shape=jax.ShapeDtypeStruct((rows, cols), x.dtype),
    )(x)


# speedup vs baseline: 49.6775x; 1.0135x over previous
import jax
import jax.numpy as jnp
from jax.experimental import pallas as pl
from jax.experimental.pallas import tpu as pltpu

_CHUNK = 512
_NBUF = 8


def _body(x_hbm, o_hbm, buf, in_sem, out_sem):
    nchunks = x_hbm.shape[0] // _CHUNK

    def in_copy(i):
        return pltpu.make_async_copy(
            x_hbm.at[pl.ds(i * _CHUNK, _CHUNK), :],
            buf.at[i % _NBUF],
            in_sem.at[i % _NBUF],
        )

    def out_copy(i):
        return pltpu.make_async_copy(
            buf.at[i % _NBUF],
            o_hbm.at[pl.ds(i * _CHUNK, _CHUNK), :],
            out_sem.at[i % _NBUF],
        )

    for i in range(min(_NBUF, nchunks)):
        in_copy(i).start()
    for i in range(nchunks):
        in_copy(i).wait()
        out_copy(i).start()
        if i + _NBUF < nchunks:
            out_copy(i).wait()
            in_copy(i + _NBUF).start()
    for i in range(max(0, nchunks - _NBUF), nchunks):
        out_copy(i).wait()


def kernel(x):
    rows, cols = x.shape
    return pl.pallas_call(
        _body,
        in_specs=[pl.BlockSpec(memory_space=pl.ANY)],
        out_specs=pl.BlockSpec(memory_space=pl.ANY),
        out_shape=jax.ShapeDtypeStruct((rows, cols), x.dtype),
        scratch_shapes=[
            pltpu.MemorySpace.VMEM((_NBUF, _CHUNK, cols), x.dtype),
            pltpu.SemaphoreType.DMA((_NBUF,)),
            pltpu.SemaphoreType.DMA((_NBUF,)),
        ],
    )(x)


# speedup vs baseline: 49.8306x; 1.0031x over previous
import jax
import jax.numpy as jnp
from jax.experimental import pallas as pl
from jax.experimental.pallas import tpu as pltpu

_CHUNK = 1024
_NBUF = 4


def _body(x_hbm, o_hbm, buf, in_sem, out_sem):
    nchunks = x_hbm.shape[0] // _CHUNK

    def in_copy(i):
        return pltpu.make_async_copy(
            x_hbm.at[pl.ds(i * _CHUNK, _CHUNK), :],
            buf.at[i % _NBUF],
            in_sem.at[i % _NBUF],
        )

    def out_copy(i):
        return pltpu.make_async_copy(
            buf.at[i % _NBUF],
            o_hbm.at[pl.ds(i * _CHUNK, _CHUNK), :],
            out_sem.at[i % _NBUF],
        )

    for i in range(min(_NBUF, nchunks)):
        in_copy(i).start()
    for i in range(nchunks):
        in_copy(i).wait()
        out_copy(i).start()
        if i + _NBUF < nchunks:
            out_copy(i).wait()
            in_copy(i + _NBUF).start()
    for i in range(max(0, nchunks - _NBUF), nchunks):
        out_copy(i).wait()


def kernel(x):
    rows, cols = x.shape
    return pl.pallas_call(
        _body,
        in_specs=[pl.BlockSpec(memory_space=pl.ANY)],
        out_specs=pl.BlockSpec(memory_space=pl.ANY),
        out_shape=jax.ShapeDtypeStruct((rows, cols), x.dtype),
        scratch_shapes=[
            pltpu.MemorySpace.VMEM((_NBUF, _CHUNK, cols), x.dtype),
            pltpu.SemaphoreType.DMA((_NBUF,)),
            pltpu.SemaphoreType.DMA((_NBUF,)),
        ],
    )(x)
